# baseline (speedup 1.0000x reference)
import functools

import jax
import jax.numpy as jnp
from jax import lax
from jax.experimental import pallas as pl
from jax.experimental.pallas import tpu as pltpu
from jax.experimental.pallas import tpu_sc as plsc

_LANES = 16
_C = 128


@functools.lru_cache(maxsize=None)
def _linear_fn(n, d_in, d_out, bn):
    def body(x_ref, wt_ref, b_ref, o_ref):
        o_ref[...] = (
            jnp.dot(x_ref[...], wt_ref[...], preferred_element_type=jnp.float32)
            + b_ref[...]
        )

    return pl.pallas_call(
        body,
        grid=(n // bn,),
        in_specs=[
            pl.BlockSpec((bn, d_in), lambda i: (i, 0)),
            pl.BlockSpec((d_in, d_out), lambda i: (0, 0)),
            pl.BlockSpec((1, d_out), lambda i: (0, 0)),
        ],
        out_specs=pl.BlockSpec((bn, d_out), lambda i: (i, 0)),
        out_shape=jax.ShapeDtypeStruct((n, d_out), jnp.float32),
    )


@functools.lru_cache(maxsize=None)
def _combine_fn(n, d, bn):
    def body(p_ref, o_ref):
        o_ref[...] = jnp.maximum(p_ref[0] + p_ref[1], 0.0)

    return pl.pallas_call(
        body,
        grid=(n // bn,),
        in_specs=[pl.BlockSpec((2, bn, d), lambda i: (0, i, 0))],
        out_specs=pl.BlockSpec((bn, d), lambda i: (i, 0)),
        out_shape=jax.ShapeDtypeStruct((n, d), jnp.float32),
    )


@functools.lru_cache(maxsize=None)
def _sc_scatter_fn(n, d, k):
    info = plsc.get_sparse_core_info()
    nc, ns = info.num_cores, info.num_subcores
    rows_per_tile = (-(-n // ns) + 7) // 8 * 8
    n_pad = ns * rows_per_tile
    mesh = plsc.VectorSubcoreMesh(core_axis_name="c", subcore_axis_name="s")

    @functools.partial(
        pl.kernel,
        mesh=mesh,
        out_type=jax.ShapeDtypeStruct((nc, n_pad, d), jnp.float32),
        scratch_types=[
            pltpu.VMEM((k, _C), jnp.int32),
            pltpu.VMEM((1, _C), jnp.int32),
            pltpu.VMEM((_C,), jnp.float32),
            pltpu.VMEM((_C, d), jnp.float32),
            pltpu.VMEM_SHARED((n_pad, d), jnp.float32),
            pltpu.SemaphoreType.DMA,
        ],
    )
    def sc_kernel(h_hbm, src_hbm, dst_hbm, w_hbm, z_hbm, out_hbm,
                  src_v, didx, wbuf, rows, acc, gsem):
        cid = lax.axis_index("c")
        sid = lax.axis_index("s")
        wid = sid * nc + cid
        pltpu.sync_copy(src_hbm.at[wid], src_v)
        base = sid * rows_per_tile
        pltpu.sync_copy(z_hbm.at[pl.ds(base, rows_per_tile)],
                        acc.at[pl.ds(base, rows_per_tile)])
        plsc.subcore_barrier()

        def chunk_body(j, carry):
            pltpu.async_copy(h_hbm.at[src_v.at[j]], rows, gsem).wait()
            pltpu.sync_copy(dst_hbm.at[wid, j], didx.at[0])
            pltpu.sync_copy(w_hbm.at[wid, j], wbuf)

            @plsc.parallel_loop(0, _C // _LANES, 1, unroll=2)
            def _(g):
                wv16 = wbuf[pl.ds(g * _LANES, _LANES)]
                for e16 in range(_LANES):
                    eidx = jnp.full((_LANES,), e16, jnp.int32)
                    wv = wv16.at[eidx].get(mode="promise_in_bounds")
                    row = g * _LANES + e16
                    for t in range(d // _LANES):
                        sl = pl.ds(t * _LANES, _LANES)
                        rows[row, sl] = rows[row, sl] * wv

            pltpu.sync_copy(rows, acc.at[didx.at[0]], add=True)
            return carry

        lax.fori_loop(0, k, chunk_body, 0)
        plsc.subcore_barrier()
        pltpu.sync_copy(acc.at[pl.ds(base, rows_per_tile)],
                        out_hbm.at[cid, pl.ds(base, rows_per_tile)])

    return sc_kernel


def kernel(x, edge_index, edge_weight, W, b):
    n, d_in = x.shape
    d_out = W.shape[0]
    e = edge_weight.shape[0]
    info = plsc.get_sparse_core_info()
    nw = info.num_cores * info.num_subcores

    h = _linear_fn(n, d_in, d_out, 1000)(x, W.T, b.reshape(1, d_out))

    k = (-(-e // (nw * _C)) + 7) // 8 * 8
    pad = nw * k * _C - e
    src = jnp.pad(edge_index[1], (0, pad)).reshape(nw, k, _C)
    dst = jnp.pad(edge_index[0], (0, pad)).reshape(nw, k, _C)
    w = jnp.pad(edge_weight, (0, pad)).reshape(nw, k, _C)
    rows_per_tile = (-(-n // info.num_subcores) + 7) // 8 * 8
    n_pad = info.num_subcores * rows_per_tile
    zeros = jnp.zeros((n_pad, d_out), jnp.float32)

    partials = _sc_scatter_fn(n, d_out, k)(h, src, dst, w, zeros)
    return _combine_fn(n, d_out, 1000)(partials[:, :n])


# speedup vs baseline: 1.2909x; 1.2909x over previous
import functools

import jax
import jax.numpy as jnp
from jax import lax
from jax.experimental import pallas as pl
from jax.experimental.pallas import tpu as pltpu
from jax.experimental.pallas import tpu_sc as plsc

_LANES = 16
_C = 128


@functools.lru_cache(maxsize=None)
def _linear_fn(n, d_in, d_out, bn):
    def body(x_ref, wt_ref, b_ref, o_ref):
        o_ref[...] = (
            jnp.dot(x_ref[...], wt_ref[...], preferred_element_type=jnp.float32)
            + b_ref[...]
        )

    return pl.pallas_call(
        body,
        grid=(n // bn,),
        in_specs=[
            pl.BlockSpec((bn, d_in), lambda i: (i, 0)),
            pl.BlockSpec((d_in, d_out), lambda i: (0, 0)),
            pl.BlockSpec((1, d_out), lambda i: (0, 0)),
        ],
        out_specs=pl.BlockSpec((bn, d_out), lambda i: (i, 0)),
        out_shape=jax.ShapeDtypeStruct((n, d_out), jnp.float32),
    )


@functools.lru_cache(maxsize=None)
def _combine_fn(n, d, bn):
    def body(p_ref, o_ref):
        o_ref[...] = jnp.maximum(p_ref[0] + p_ref[1], 0.0)

    return pl.pallas_call(
        body,
        grid=(n // bn,),
        in_specs=[pl.BlockSpec((2, bn, d), lambda i: (0, i, 0))],
        out_specs=pl.BlockSpec((bn, d), lambda i: (i, 0)),
        out_shape=jax.ShapeDtypeStruct((n, d), jnp.float32),
    )


@functools.lru_cache(maxsize=None)
def _sc_scatter_fn(n, d, k):
    info = plsc.get_sparse_core_info()
    nc, ns = info.num_cores, info.num_subcores
    rows_per_tile = (-(-n // ns) + 7) // 8 * 8
    n_pad = ns * rows_per_tile
    mesh = plsc.VectorSubcoreMesh(core_axis_name="c", subcore_axis_name="s")

    @functools.partial(
        pl.kernel,
        mesh=mesh,
        out_type=jax.ShapeDtypeStruct((nc, n_pad, d), jnp.float32),
        scratch_types=[
            pltpu.VMEM((k, _C), jnp.int32),
            pltpu.VMEM((2, _C), jnp.int32),
            pltpu.VMEM((2, _C), jnp.float32),
            pltpu.VMEM((_C, d), jnp.float32),
            pltpu.VMEM((_C, d), jnp.float32),
            pltpu.VMEM_SHARED((n_pad, d), jnp.float32),
            pltpu.SemaphoreType.DMA,
            pltpu.SemaphoreType.DMA,
        ],
    )
    def sc_kernel(h_hbm, src_hbm, dst_hbm, w_hbm, z_hbm, out_hbm,
                  src_v, didx, wall, rows0, rows1, acc, gsem0, gsem1):
        cid = lax.axis_index("c")
        sid = lax.axis_index("s")
        wid = sid * nc + cid
        pltpu.sync_copy(src_hbm.at[wid], src_v)
        base = sid * rows_per_tile
        pltpu.sync_copy(z_hbm.at[pl.ds(base, rows_per_tile)],
                        acc.at[pl.ds(base, rows_per_tile)])
        plsc.subcore_barrier()

        def scale(slot, rows):
            @plsc.parallel_loop(0, _C // _LANES, 1, unroll=2)
            def _(g):
                wv16 = wall[slot, pl.ds(g * _LANES, _LANES)]
                for e16 in range(_LANES):
                    eidx = jnp.full((_LANES,), e16, jnp.int32)
                    wv = wv16.at[eidx].get(mode="promise_in_bounds")
                    row = g * _LANES + e16
                    for t in range(d // _LANES):
                        sl = pl.ds(t * _LANES, _LANES)
                        rows[row, sl] = rows[row, sl] * wv

        pltpu.async_copy(h_hbm.at[src_v.at[0]], rows0, gsem0)
        pltpu.async_copy(h_hbm.at[src_v.at[1]], rows1, gsem1)

        def pair_body(m, carry):
            j0 = 2 * m
            j1 = j0 + 1
            pltpu.sync_copy(dst_hbm.at[wid, j0], didx.at[0])
            pltpu.sync_copy(w_hbm.at[wid, j0], wall.at[0])
            pltpu.make_async_copy(h_hbm.at[src_v.at[j0]], rows0, gsem0).wait()
            scale(0, rows0)
            pltpu.sync_copy(rows0, acc.at[didx.at[0]], add=True)
            jn0 = jnp.minimum(j0 + 2, k - 1)
            pltpu.async_copy(h_hbm.at[src_v.at[jn0]], rows0, gsem0)
            pltpu.sync_copy(dst_hbm.at[wid, j1], didx.at[1])
            pltpu.sync_copy(w_hbm.at[wid, j1], wall.at[1])
            pltpu.make_async_copy(h_hbm.at[src_v.at[j1]], rows1, gsem1).wait()
            scale(1, rows1)
            pltpu.sync_copy(rows1, acc.at[didx.at[1]], add=True)
            jn1 = jnp.minimum(j1 + 2, k - 1)
            pltpu.async_copy(h_hbm.at[src_v.at[jn1]], rows1, gsem1)
            return carry

        lax.fori_loop(0, k // 2, pair_body, 0)
        jl = k - 1
        pltpu.make_async_copy(h_hbm.at[src_v.at[jl]], rows0, gsem0).wait()
        pltpu.make_async_copy(h_hbm.at[src_v.at[jl]], rows1, gsem1).wait()
        plsc.subcore_barrier()
        pltpu.sync_copy(acc.at[pl.ds(base, rows_per_tile)],
                        out_hbm.at[cid, pl.ds(base, rows_per_tile)])

    return sc_kernel


def kernel(x, edge_index, edge_weight, W, b):
    n, d_in = x.shape
    d_out = W.shape[0]
    e = edge_weight.shape[0]
    info = plsc.get_sparse_core_info()
    nw = info.num_cores * info.num_subcores

    h = _linear_fn(n, d_in, d_out, 1000)(x, W.T, b.reshape(1, d_out))

    k = (-(-e // (nw * _C)) + 7) // 8 * 8
    pad = nw * k * _C - e
    src = jnp.pad(edge_index[1], (0, pad)).reshape(nw, k, _C)
    dst = jnp.pad(edge_index[0], (0, pad)).reshape(nw, k, _C)
    w = jnp.pad(edge_weight, (0, pad)).reshape(nw, k, _C)
    rows_per_tile = (-(-n // info.num_subcores) + 7) // 8 * 8
    n_pad = info.num_subcores * rows_per_tile
    zeros = jnp.zeros((n_pad, d_out), jnp.float32)

    partials = _sc_scatter_fn(n, d_out, k)(h, src, dst, w, zeros)
    return _combine_fn(n, d_out, 1000)(partials[:, :n])


# speedup vs baseline: 1.3063x; 1.0119x over previous
import functools

import jax
import jax.numpy as jnp
from jax import lax
from jax.experimental import pallas as pl
from jax.experimental.pallas import tpu as pltpu
from jax.experimental.pallas import tpu_sc as plsc

_LANES = 16
_C = 128


@functools.lru_cache(maxsize=None)
def _linear_fn(n, d_in, d_out, bn):
    def body(x_ref, wt_ref, b_ref, o_ref):
        o_ref[...] = (
            jnp.dot(x_ref[...], wt_ref[...], preferred_element_type=jnp.float32)
            + b_ref[...]
        )

    return pl.pallas_call(
        body,
        grid=(n // bn,),
        in_specs=[
            pl.BlockSpec((bn, d_in), lambda i: (i, 0)),
            pl.BlockSpec((d_in, d_out), lambda i: (0, 0)),
            pl.BlockSpec((1, d_out), lambda i: (0, 0)),
        ],
        out_specs=pl.BlockSpec((bn, d_out), lambda i: (i, 0)),
        out_shape=jax.ShapeDtypeStruct((n, d_out), jnp.float32),
    )


@functools.lru_cache(maxsize=None)
def _combine_fn(n, d, bn):
    def body(p_ref, o_ref):
        o_ref[...] = jnp.maximum(p_ref[0] + p_ref[1], 0.0)

    return pl.pallas_call(
        body,
        grid=(n // bn,),
        in_specs=[pl.BlockSpec((2, bn, d), lambda i: (0, i, 0))],
        out_specs=pl.BlockSpec((bn, d), lambda i: (i, 0)),
        out_shape=jax.ShapeDtypeStruct((n, d), jnp.float32),
    )


@functools.lru_cache(maxsize=None)
def _sc_scatter_fn(n, d, k):
    info = plsc.get_sparse_core_info()
    nc, ns = info.num_cores, info.num_subcores
    rows_per_tile = (-(-n // ns) + 7) // 8 * 8
    n_pad = ns * rows_per_tile
    mesh = plsc.VectorSubcoreMesh(core_axis_name="c", subcore_axis_name="s")

    @functools.partial(
        pl.kernel,
        mesh=mesh,
        out_type=jax.ShapeDtypeStruct((nc, n_pad, d), jnp.float32),
        scratch_types=[
            pltpu.VMEM((k, _C), jnp.int32),
            pltpu.VMEM((2, _C), jnp.int32),
            pltpu.VMEM((2, _C), jnp.float32),
            pltpu.VMEM((_C, d), jnp.float32),
            pltpu.VMEM((_C, d), jnp.float32),
            pltpu.VMEM_SHARED((n_pad, d), jnp.float32),
            pltpu.SemaphoreType.DMA,
            pltpu.SemaphoreType.DMA,
        ],
    )
    def sc_kernel(h_hbm, src_hbm, dst_hbm, w_hbm, z_hbm, out_hbm,
                  src_v, didx, wall, rows0, rows1, acc, gsem0, gsem1):
        cid = lax.axis_index("c")
        sid = lax.axis_index("s")
        wid = sid * nc + cid
        pltpu.sync_copy(src_hbm.at[wid], src_v)
        base = sid * rows_per_tile
        pltpu.sync_copy(z_hbm.at[pl.ds(base, rows_per_tile)],
                        acc.at[pl.ds(base, rows_per_tile)])
        plsc.subcore_barrier()

        def scale(slot, rows):
            @plsc.parallel_loop(0, _C // _LANES, 1, unroll=2)
            def _(g):
                wv16 = wall[slot, pl.ds(g * _LANES, _LANES)]
                for e16 in range(_LANES):
                    eidx = jnp.full((_LANES,), e16, jnp.int32)
                    wv = wv16.at[eidx].get(mode="promise_in_bounds")
                    row = g * _LANES + e16
                    for t in range(d // _LANES):
                        sl = pl.ds(t * _LANES, _LANES)
                        rows[row, sl] = rows[row, sl] * wv

        def prefetch(slot, rows, sem, j):
            pltpu.async_copy(dst_hbm.at[wid, j], didx.at[slot], sem)
            pltpu.async_copy(w_hbm.at[wid, j], wall.at[slot], sem)
            pltpu.async_copy(h_hbm.at[src_v.at[j]], rows, sem)

        def drain(slot, rows, sem, j):
            pltpu.make_async_copy(dst_hbm.at[wid, j], didx.at[slot], sem).wait()
            pltpu.make_async_copy(w_hbm.at[wid, j], wall.at[slot], sem).wait()
            pltpu.make_async_copy(h_hbm.at[src_v.at[j]], rows, sem).wait()

        prefetch(0, rows0, gsem0, 0)
        prefetch(1, rows1, gsem1, 1)

        def pair_body(m, carry):
            j0 = 2 * m
            j1 = j0 + 1
            drain(0, rows0, gsem0, j0)
            scale(0, rows0)
            pltpu.sync_copy(rows0, acc.at[didx.at[0]], add=True)
            prefetch(0, rows0, gsem0, jnp.minimum(j0 + 2, k - 1))
            drain(1, rows1, gsem1, j1)
            scale(1, rows1)
            pltpu.sync_copy(rows1, acc.at[didx.at[1]], add=True)
            prefetch(1, rows1, gsem1, jnp.minimum(j1 + 2, k - 1))
            return carry

        lax.fori_loop(0, k // 2, pair_body, 0)
        jl = k - 1
        drain(0, rows0, gsem0, jl)
        drain(1, rows1, gsem1, jl)
        plsc.subcore_barrier()
        pltpu.sync_copy(acc.at[pl.ds(base, rows_per_tile)],
                        out_hbm.at[cid, pl.ds(base, rows_per_tile)])

    return sc_kernel


def kernel(x, edge_index, edge_weight, W, b):
    n, d_in = x.shape
    d_out = W.shape[0]
    e = edge_weight.shape[0]
    info = plsc.get_sparse_core_info()
    nw = info.num_cores * info.num_subcores

    h = _linear_fn(n, d_in, d_out, 1000)(x, W.T, b.reshape(1, d_out))

    k = (-(-e // (nw * _C)) + 7) // 8 * 8
    pad = nw * k * _C - e
    src = jnp.pad(edge_index[1], (0, pad)).reshape(nw, k, _C)
    dst = jnp.pad(edge_index[0], (0, pad)).reshape(nw, k, _C)
    w = jnp.pad(edge_weight, (0, pad)).reshape(nw, k, _C)
    rows_per_tile = (-(-n // info.num_subcores) + 7) // 8 * 8
    n_pad = info.num_subcores * rows_per_tile
    zeros = jnp.zeros((n_pad, d_out), jnp.float32)

    partials = _sc_scatter_fn(n, d_out, k)(h, src, dst, w, zeros)
    return _combine_fn(n, d_out, 1000)(partials[:, :n])


# speedup vs baseline: 1.3290x; 1.0174x over previous
import functools

import jax
import jax.numpy as jnp
from jax import lax
from jax.experimental import pallas as pl
from jax.experimental.pallas import tpu as pltpu
from jax.experimental.pallas import tpu_sc as plsc

_LANES = 16
_C = 128


@functools.lru_cache(maxsize=None)
def _linear_fn(n, d_in, d_out, bn):
    def body(x_ref, wt_ref, b_ref, o_ref):
        o_ref[...] = (
            jnp.dot(x_ref[...], wt_ref[...], preferred_element_type=jnp.float32)
            + b_ref[...]
        )

    return pl.pallas_call(
        body,
        grid=(n // bn,),
        in_specs=[
            pl.BlockSpec((bn, d_in), lambda i: (i, 0)),
            pl.BlockSpec((d_in, d_out), lambda i: (0, 0)),
            pl.BlockSpec((1, d_out), lambda i: (0, 0)),
        ],
        out_specs=pl.BlockSpec((bn, d_out), lambda i: (i, 0)),
        out_shape=jax.ShapeDtypeStruct((n, d_out), jnp.float32),
    )


@functools.lru_cache(maxsize=None)
def _combine_fn(n, d, bn):
    def body(p_ref, o_ref):
        o_ref[...] = jnp.maximum(p_ref[0] + p_ref[1], 0.0)

    return pl.pallas_call(
        body,
        grid=(n // bn,),
        in_specs=[pl.BlockSpec((2, bn, d), lambda i: (0, i, 0))],
        out_specs=pl.BlockSpec((bn, d), lambda i: (i, 0)),
        out_shape=jax.ShapeDtypeStruct((n, d), jnp.float32),
    )


@functools.lru_cache(maxsize=None)
def _sc_scatter_fn(n, d, k):
    info = plsc.get_sparse_core_info()
    nc, ns = info.num_cores, info.num_subcores
    rows_per_tile = (-(-n // ns) + 7) // 8 * 8
    n_pad = ns * rows_per_tile
    mesh = plsc.VectorSubcoreMesh(core_axis_name="c", subcore_axis_name="s")

    @functools.partial(
        pl.kernel,
        mesh=mesh,
        out_type=jax.ShapeDtypeStruct((nc, n_pad, d), jnp.float32),
        scratch_types=[
            pltpu.VMEM((k, _C), jnp.int32),
            pltpu.VMEM((2, _C), jnp.int32),
            pltpu.VMEM((2, _C), jnp.float32),
            pltpu.VMEM((_C, d), jnp.float32),
            pltpu.VMEM((_C, d), jnp.float32),
            pltpu.VMEM_SHARED((n_pad, d), jnp.float32),
            pltpu.SemaphoreType.DMA,
            pltpu.SemaphoreType.DMA,
        ],
    )
    def sc_kernel(h_hbm, src_hbm, dst_hbm, w_hbm, z_hbm, out_hbm,
                  src_v, didx, wall, rows0, rows1, acc, gsem0, gsem1):
        cid = lax.axis_index("c")
        sid = lax.axis_index("s")
        wid = sid * nc + cid
        pltpu.sync_copy(src_hbm.at[wid], src_v)
        base = sid * rows_per_tile
        pltpu.sync_copy(z_hbm.at[pl.ds(base, rows_per_tile)],
                        acc.at[pl.ds(base, rows_per_tile)])
        plsc.subcore_barrier()

        def scale(slot, rows):
            @plsc.parallel_loop(0, _C // _LANES, 1, unroll=4)
            def _(g):
                wv16 = wall[slot, pl.ds(g * _LANES, _LANES)]
                for e16 in range(_LANES):
                    eidx = jnp.full((_LANES,), e16, jnp.int32)
                    wv = wv16.at[eidx].get(mode="promise_in_bounds")
                    row = g * _LANES + e16
                    for t in range(d // _LANES):
                        sl = pl.ds(t * _LANES, _LANES)
                        rows[row, sl] = rows[row, sl] * wv

        def prefetch(slot, rows, sem, j):
            pltpu.async_copy(dst_hbm.at[wid, j], didx.at[slot], sem)
            pltpu.async_copy(w_hbm.at[wid, j], wall.at[slot], sem)
            pltpu.async_copy(h_hbm.at[src_v.at[j]], rows, sem)

        def drain(slot, rows, sem, j):
            pltpu.make_async_copy(dst_hbm.at[wid, j], didx.at[slot], sem).wait()
            pltpu.make_async_copy(w_hbm.at[wid, j], wall.at[slot], sem).wait()
            pltpu.make_async_copy(h_hbm.at[src_v.at[j]], rows, sem).wait()

        prefetch(0, rows0, gsem0, 0)
        prefetch(1, rows1, gsem1, 1)

        def pair_body(m, carry):
            j0 = 2 * m
            j1 = j0 + 1
            drain(0, rows0, gsem0, j0)
            scale(0, rows0)
            pltpu.sync_copy(rows0, acc.at[didx.at[0]], add=True)
            prefetch(0, rows0, gsem0, jnp.minimum(j0 + 2, k - 1))
            drain(1, rows1, gsem1, j1)
            scale(1, rows1)
            pltpu.sync_copy(rows1, acc.at[didx.at[1]], add=True)
            prefetch(1, rows1, gsem1, jnp.minimum(j1 + 2, k - 1))
            return carry

        lax.fori_loop(0, k // 2, pair_body, 0)
        jl = k - 1
        drain(0, rows0, gsem0, jl)
        drain(1, rows1, gsem1, jl)
        plsc.subcore_barrier()
        pltpu.sync_copy(acc.at[pl.ds(base, rows_per_tile)],
                        out_hbm.at[cid, pl.ds(base, rows_per_tile)])

    return sc_kernel


def kernel(x, edge_index, edge_weight, W, b):
    n, d_in = x.shape
    d_out = W.shape[0]
    e = edge_weight.shape[0]
    info = plsc.get_sparse_core_info()
    nw = info.num_cores * info.num_subcores

    h = _linear_fn(n, d_in, d_out, 1000)(x, W.T, b.reshape(1, d_out))

    k = (-(-e // (nw * _C)) + 7) // 8 * 8
    pad = nw * k * _C - e
    src = jnp.pad(edge_index[1], (0, pad)).reshape(nw, k, _C)
    dst = jnp.pad(edge_index[0], (0, pad)).reshape(nw, k, _C)
    w = jnp.pad(edge_weight, (0, pad)).reshape(nw, k, _C)
    rows_per_tile = (-(-n // info.num_subcores) + 7) // 8 * 8
    n_pad = info.num_subcores * rows_per_tile
    zeros = jnp.zeros((n_pad, d_out), jnp.float32)

    partials = _sc_scatter_fn(n, d_out, k)(h, src, dst, w, zeros)
    return _combine_fn(n, d_out, 1000)(partials[:, :n])
